# baseline (speedup 1.0000x reference)
import functools

import jax
import jax.numpy as jnp
from jax import lax
from jax.experimental import pallas as pl
from jax.experimental.pallas import tpu as pltpu
from jax.experimental.pallas import tpu_sc as plsc

_H = 128
_W = 128
_B = 2
_N = 1024
_P = 32
_QMAX = 30.0

_mesh = plsc.VectorSubcoreMesh(core_axis_name="c", subcore_axis_name="s")


@functools.partial(
    pl.kernel,
    out_type=jax.ShapeDtypeStruct((32, 4 * _P * _P), jnp.float32),
    mesh=_mesh,
    scratch_types=[
        pltpu.VMEM((_N * 16,), jnp.float32),
        pltpu.VMEM((5 * _N,), jnp.float32),
        pltpu.SMEM((_N + 1,), jnp.int32),
        pltpu.VMEM((16,), jnp.int32),
        pltpu.VMEM((_P * _P,), jnp.float32),
        pltpu.VMEM((4 * _P * _P,), jnp.float32),
    ],
)
def _sc_render(pk_hbm, cull_hbm, out_hbm, pk, cull, hits, hvbuf, t, acc):
    wid = lax.axis_index("s") * 2 + lax.axis_index("c")
    b = wid // 16
    pidx = wid % 16
    pr0 = (pidx // 4) * _P
    pc0 = (pidx % 4) * _P
    pr0f = pr0.astype(jnp.float32)
    pc0f = pc0.astype(jnp.float32)
    myb = b.astype(jnp.float32)

    pltpu.sync_copy(pk_hbm, pk)
    pltpu.sync_copy(cull_hbm, cull)

    ones = jnp.full((16,), 1.0, jnp.float32)
    zeros = jnp.zeros((16,), jnp.float32)

    def init_t(j, c):
        t[pl.ds(j * 16, 16)] = ones
        return c

    lax.fori_loop(0, (_P * _P) // 16, init_t, 0)

    def init_acc(j, c):
        acc[pl.ds(j * 16, 16)] = zeros
        return c

    lax.fori_loop(0, (4 * _P * _P) // 16, init_acc, 0)

    lanei = lax.iota(jnp.int32, 16)


    lane = lanei.astype(jnp.float32) + 0.5
    x0v = lane + pc0f
    x1v = x0v + 16.0

    def cull_body(j, cnt):
        j16 = j * 16
        bidv = cull[pl.ds(j16, 16)]
        myv = cull[pl.ds(j16 + _N, 16)]
        ryv = cull[pl.ds(j16 + 2 * _N, 16)]
        mxv = cull[pl.ds(j16 + 3 * _N, 16)]
        rxv = cull[pl.ds(j16 + 4 * _N, 16)]
        hit = bidv == myb
        hit = jnp.logical_and(hit, myv - ryv - 0.5 <= pr0f + (_P - 1.0))
        hit = jnp.logical_and(hit, myv + ryv + 0.5 >= pr0f)
        hit = jnp.logical_and(hit, mxv - rxv - 0.5 < pc0f + float(_P))
        hit = jnp.logical_and(hit, mxv + rxv + 0.5 > pc0f)
        one16 = jnp.full((16,), 1, jnp.int32)
        zero16 = jnp.full((16,), 0, jnp.int32)
        hvbuf[pl.ds(0, 16)] = jnp.where(hit, one16, zero16)
        hb = hvbuf[pl.ds(0, 16)]
        base = j * 16
        for l in range(16):
            hits[cnt] = base + l
            cnt = cnt + hb[l]
        return cnt

    cnt = lax.fori_loop(0, _N // 16, cull_body, 0)

    def hit_body(k, c):
        i = hits[k]
        v = pk[pl.ds(i * 16, 16)]
        mx = v[1]
        my = v[2]
        a = v[3]
        bc = v[4]
        d = v[5]
        cr = v[6]
        cg = v[7]
        cb = v[8]
        lnop = v[9]
        ry = v[10]
        rx = v[11]
        ylo = jnp.maximum(my - ry - 0.5, pr0f).astype(jnp.int32)
        yhi = jnp.minimum(my + ry + 0.5, pr0f + (_P - 1.0)).astype(jnp.int32) + 1
        inc0 = mx - rx - 0.5 < pc0f + 16.0
        inc1 = mx + rx + 0.5 > pc0f + 16.0

        def make_row(xv, off):
            dxh = xv - mx
            u = a * dxh * dxh
            vv = bc * dxh

            def row(r, rc):
                dy = r.astype(jnp.float32) + 0.5 - my
                ddy2 = d * dy * dy + lnop
                q = (vv * dy + ddy2) + u
                e = jnp.exp(q)
                bb = (r - pr0) * _P + off
                tt = t[pl.ds(bb, 16)]
                w = e * tt
                t[pl.ds(bb, 16)] = tt - w
                plsc.addupdate(acc.at[pl.ds(bb, 16)], cr * w)
                plsc.addupdate(acc.at[pl.ds(bb + 1024, 16)], cg * w)
                plsc.addupdate(acc.at[pl.ds(bb + 2048, 16)], cb * w)
                plsc.addupdate(acc.at[pl.ds(bb + 3072, 16)], w)
                return rc

            return row

        @pl.when(inc0)
        def _():
            body0 = make_row(x0v, 0)

            @plsc.parallel_loop(ylo, yhi, unroll=4)
            def _(r):
                body0(r, 0)

        @pl.when(inc1)
        def _():
            body1 = make_row(x1v, 16)

            @plsc.parallel_loop(ylo, yhi, unroll=4)
            def _(r):
                body1(r, 0)

        return c

    lax.fori_loop(0, cnt, hit_body, 0)

    pltpu.sync_copy(acc, out_hbm.at[wid])


@jax.jit
def kernel(batch_ids, means, sigmas, rhos, colors, opacs):
    c = jnp.cos(rhos[:, 0])
    s = jnp.sin(rhos[:, 0])
    s1 = sigmas[:, 0]
    s2 = sigmas[:, 1]
    i1 = 1.0 / s1
    i2 = 1.0 / s2
    a = -0.5 * (c * c * i1 + s * s * i2)
    d = -0.5 * (s * s * i1 + c * c * i2)
    bc = -0.5 * (2.0 * c * s * (i1 - i2))
    sxx = c * c * s1 + s * s * s2
    syy = s * s * s1 + c * c * s2
    rx = jnp.sqrt(_QMAX * sxx)
    ry = jnp.sqrt(_QMAX * syy)
    bidf = batch_ids.astype(jnp.float32)
    lnop = jnp.log(opacs[:, 0])
    z = jnp.zeros_like(a)
    pk = jnp.stack([
        bidf, means[:, 0], means[:, 1],
        a, bc, d, colors[:, 0], colors[:, 1], colors[:, 2],
        lnop, ry, rx, z, z, z, z,
    ], axis=1).reshape(-1)
    cull = jnp.concatenate([bidf, means[:, 1], ry, means[:, 0], rx])
    out = _sc_render(pk, cull)
    out = out.reshape(_B, 4, 4, 4, _P, _P)
    out = out.transpose(0, 3, 1, 4, 2, 5).reshape(_B, 4, _H, _W)
    return out


# speedup vs baseline: 1.0982x; 1.0982x over previous
import functools

import jax
import jax.numpy as jnp
from jax import lax
from jax.experimental import pallas as pl
from jax.experimental.pallas import tpu as pltpu
from jax.experimental.pallas import tpu_sc as plsc

_H = 128
_W = 128
_B = 2
_N = 1024
_P = 32
_QMAX = 30.0

_mesh = plsc.VectorSubcoreMesh(core_axis_name="c", subcore_axis_name="s")


@functools.partial(
    pl.kernel,
    out_type=jax.ShapeDtypeStruct((32, 4 * _P * _P), jnp.float32),
    mesh=_mesh,
    scratch_types=[
        pltpu.VMEM((_N * 16,), jnp.float32),
        pltpu.VMEM((5 * _N,), jnp.float32),
        pltpu.SMEM((_N + 1,), jnp.int32),
        pltpu.VMEM((16,), jnp.int32),
        pltpu.VMEM((_P * _P,), jnp.float32),
        pltpu.VMEM((4 * _P * _P,), jnp.float32),
    ],
)
def _sc_render(pk_hbm, cull_hbm, out_hbm, pk, cull, hits, hvbuf, t, acc):
    wid = lax.axis_index("s") * 2 + lax.axis_index("c")
    b = wid // 16
    pidx = wid % 16
    pr0 = (pidx // 4) * _P
    pc0 = (pidx % 4) * _P
    pr0f = pr0.astype(jnp.float32)
    pc0f = pc0.astype(jnp.float32)
    myb = b.astype(jnp.float32)

    pltpu.sync_copy(pk_hbm, pk)
    pltpu.sync_copy(cull_hbm, cull)

    ones = jnp.full((16,), 1.0, jnp.float32)
    zeros = jnp.zeros((16,), jnp.float32)

    def init_t(j, c):
        t[pl.ds(j * 16, 16)] = ones
        return c

    lax.fori_loop(0, (_P * _P) // 16, init_t, 0)

    def init_acc(j, c):
        acc[pl.ds(j * 16, 16)] = zeros
        return c

    lax.fori_loop(0, (4 * _P * _P) // 16, init_acc, 0)

    lanei = lax.iota(jnp.int32, 16)


    lane = lanei.astype(jnp.float32) + 0.5
    x0v = lane + pc0f
    x1v = x0v + 16.0

    def cull_body(j, cnt):
        j16 = j * 16
        bidv = cull[pl.ds(j16, 16)]
        myv = cull[pl.ds(j16 + _N, 16)]
        ryv = cull[pl.ds(j16 + 2 * _N, 16)]
        mxv = cull[pl.ds(j16 + 3 * _N, 16)]
        rxv = cull[pl.ds(j16 + 4 * _N, 16)]
        hit = bidv == myb
        hit = jnp.logical_and(hit, myv - ryv - 0.5 <= pr0f + (_P - 1.0))
        hit = jnp.logical_and(hit, myv + ryv + 0.5 >= pr0f)
        hit = jnp.logical_and(hit, mxv - rxv - 0.5 < pc0f + float(_P))
        hit = jnp.logical_and(hit, mxv + rxv + 0.5 > pc0f)
        one16 = jnp.full((16,), 1, jnp.int32)
        zero16 = jnp.full((16,), 0, jnp.int32)
        hvbuf[pl.ds(0, 16)] = jnp.where(hit, one16, zero16)
        hb = hvbuf[pl.ds(0, 16)]
        base = j * 16
        for l in range(16):
            hits[cnt] = base + l
            cnt = cnt + hb[l]
        return cnt

    cnt = lax.fori_loop(0, _N // 16, cull_body, 0)

    def hit_body(k, c):
        i = hits[k]
        v = pk[pl.ds(i * 16, 16)]
        mx = v[1]
        my = v[2]
        a = v[3]
        bc = v[4]
        d = v[5]
        cr = v[6]
        cg = v[7]
        cb = v[8]
        lnop = v[9]
        ry = v[10]
        rx = v[11]
        ylo = jnp.maximum(my - ry - 0.5, pr0f).astype(jnp.int32)
        yhi = jnp.minimum(my + ry + 0.5, pr0f + (_P - 1.0)).astype(jnp.int32) + 1
        inc0 = mx - rx - 0.5 < pc0f + 16.0
        inc1 = mx + rx + 0.5 > pc0f + 16.0

        dx0 = x0v - mx
        u0 = a * dx0 * dx0
        vv0 = bc * dx0
        dx1 = x1v - mx
        u1 = a * dx1 * dx1
        vv1 = bc * dx1

        def half_step(r, off, u, vv, dy, ddy2):
            q = (vv * dy + ddy2) + u
            e = jnp.exp(q)
            bb = (r - pr0) * _P + off
            tt = t[pl.ds(bb, 16)]
            w = e * tt
            t[pl.ds(bb, 16)] = tt - w
            plsc.addupdate(acc.at[pl.ds(bb, 16)], cr * w)
            plsc.addupdate(acc.at[pl.ds(bb + 1024, 16)], cg * w)
            plsc.addupdate(acc.at[pl.ds(bb + 2048, 16)], cb * w)
            plsc.addupdate(acc.at[pl.ds(bb + 3072, 16)], w)

        @pl.when(jnp.logical_and(inc0, inc1))
        def _():
            @plsc.parallel_loop(ylo, yhi, unroll=2)
            def _(r):
                dy = r.astype(jnp.float32) + 0.5 - my
                ddy2 = d * dy * dy + lnop
                half_step(r, 0, u0, vv0, dy, ddy2)
                half_step(r, 16, u1, vv1, dy, ddy2)

        @pl.when(jnp.logical_and(inc0, jnp.logical_not(inc1)))
        def _():
            @plsc.parallel_loop(ylo, yhi, unroll=2)
            def _(r):
                dy = r.astype(jnp.float32) + 0.5 - my
                ddy2 = d * dy * dy + lnop
                half_step(r, 0, u0, vv0, dy, ddy2)

        @pl.when(jnp.logical_and(inc1, jnp.logical_not(inc0)))
        def _():
            @plsc.parallel_loop(ylo, yhi, unroll=2)
            def _(r):
                dy = r.astype(jnp.float32) + 0.5 - my
                ddy2 = d * dy * dy + lnop
                half_step(r, 16, u1, vv1, dy, ddy2)

        return c

    lax.fori_loop(0, cnt, hit_body, 0)

    pltpu.sync_copy(acc, out_hbm.at[wid])


@jax.jit
def kernel(batch_ids, means, sigmas, rhos, colors, opacs):
    c = jnp.cos(rhos[:, 0])
    s = jnp.sin(rhos[:, 0])
    s1 = sigmas[:, 0]
    s2 = sigmas[:, 1]
    i1 = 1.0 / s1
    i2 = 1.0 / s2
    a = -0.5 * (c * c * i1 + s * s * i2)
    d = -0.5 * (s * s * i1 + c * c * i2)
    bc = -0.5 * (2.0 * c * s * (i1 - i2))
    sxx = c * c * s1 + s * s * s2
    syy = s * s * s1 + c * c * s2
    rx = jnp.sqrt(_QMAX * sxx)
    ry = jnp.sqrt(_QMAX * syy)
    bidf = batch_ids.astype(jnp.float32)
    lnop = jnp.log(opacs[:, 0])
    z = jnp.zeros_like(a)
    pk = jnp.stack([
        bidf, means[:, 0], means[:, 1],
        a, bc, d, colors[:, 0], colors[:, 1], colors[:, 2],
        lnop, ry, rx, z, z, z, z,
    ], axis=1).reshape(-1)
    cull = jnp.concatenate([bidf, means[:, 1], ry, means[:, 0], rx])
    out = _sc_render(pk, cull)
    out = out.reshape(_B, 4, 4, 4, _P, _P)
    out = out.transpose(0, 3, 1, 4, 2, 5).reshape(_B, 4, _H, _W)
    return out


# speedup vs baseline: 1.3398x; 1.2200x over previous
import functools

import jax
import jax.numpy as jnp
from jax import lax
from jax.experimental import pallas as pl
from jax.experimental.pallas import tpu as pltpu
from jax.experimental.pallas import tpu_sc as plsc

_H = 128
_W = 128
_B = 2
_N = 1024
_PR = 16
_PC = 32
_PPX = _PR * _PC
_QMAX = 30.0
_ROWS = 48
_LOG2E = 1.4426950408889634

_mesh = plsc.VectorSubcoreMesh(core_axis_name="c", subcore_axis_name="s")


@functools.partial(
    pl.kernel,
    out_type=jax.ShapeDtypeStruct((32, 4 * _PPX), jnp.float32),
    mesh=_mesh,
    scratch_types=[
        pltpu.VMEM((_N * 16,), jnp.float32),
        pltpu.VMEM((5 * _N,), jnp.float32),
        pltpu.SMEM((_N + 1,), jnp.int32),
        pltpu.VMEM((16,), jnp.int32),
        pltpu.VMEM((_PPX,), jnp.float32),
        pltpu.VMEM((4 * _PPX,), jnp.float32),
    ],
)
def _sc_render(pk_hbm, cull_hbm, out_hbm, pk, cull, hits, hvbuf, t, acc):
    wid = lax.axis_index("s") * 2 + lax.axis_index("c")
    pr0 = (wid // 4) * _PR
    pc0 = (wid % 4) * _PC
    pr0f = pr0.astype(jnp.float32)
    pc0f = pc0.astype(jnp.float32)

    pltpu.sync_copy(pk_hbm, pk)
    pltpu.sync_copy(cull_hbm, cull)

    ones = jnp.full((16,), 1.0, jnp.float32)
    zeros = jnp.zeros((16,), jnp.float32)

    def init_t(j, c):
        t[pl.ds(j * 16, 16)] = ones
        return c

    lax.fori_loop(0, _PPX // 16, init_t, 0)

    def init_acc(j, c):
        acc[pl.ds(j * 16, 16)] = zeros
        return c

    lax.fori_loop(0, (4 * _PPX) // 16, init_acc, 0)

    lanei = lax.iota(jnp.int32, 16)
    lane = lanei.astype(jnp.float32) + 0.5
    x0v = lane + pc0f
    x1v = x0v + 16.0

    def cull_body(j, cnt):
        j16 = j * 16
        bidv = cull[pl.ds(j16, 16)]
        myv = cull[pl.ds(j16 + _N, 16)]
        ryv = cull[pl.ds(j16 + 2 * _N, 16)]
        mxv = cull[pl.ds(j16 + 3 * _N, 16)]
        rxv = cull[pl.ds(j16 + 4 * _N, 16)]
        hit = bidv == 1.0
        hit = jnp.logical_and(hit, myv - ryv - 0.5 <= pr0f + (_PR - 1.0))
        hit = jnp.logical_and(hit, myv + ryv + 0.5 >= pr0f)
        hit = jnp.logical_and(hit, mxv - rxv - 0.5 < pc0f + float(_PC))
        hit = jnp.logical_and(hit, mxv + rxv + 0.5 > pc0f)
        one16 = jnp.full((16,), 1, jnp.int32)
        zero16 = jnp.full((16,), 0, jnp.int32)
        hvbuf[pl.ds(0, 16)] = jnp.where(hit, one16, zero16)
        hb = hvbuf[pl.ds(0, 16)]
        base = j * 16
        for l in range(16):
            hits[cnt] = base + l
            cnt = cnt + hb[l]
        return cnt

    cnt = lax.fori_loop(0, _N // 16, cull_body, 0)

    def hit_body(k, c):
        i = hits[k]
        v = pk[pl.ds(i * 16, 16)]
        mx = v[1]
        my = v[2]
        a = v[3]
        bc = v[4]
        d = v[5]
        cr = v[6]
        cg = v[7]
        cb = v[8]
        lnop = v[9]
        ry = v[10]
        rx = v[11]
        ylo = jnp.maximum(my - ry - 0.5, pr0f).astype(jnp.int32)
        yhi = jnp.minimum(my + ry + 0.5, pr0f + (_PR - 1.0)).astype(jnp.int32) + 1
        inc0 = mx - rx - 0.5 < pc0f + 16.0
        inc1 = mx + rx + 0.5 > pc0f + 16.0

        dx0 = x0v - mx
        u0 = a * dx0 * dx0
        vv0 = bc * dx0
        dx1 = x1v - mx
        u1 = a * dx1 * dx1
        vv1 = bc * dx1

        def half_step(r, off, u, vv, dy, ddy2):
            q = (vv * dy + ddy2) + u
            e = jnp.exp(q)
            bb = (r - pr0) * _PC + off
            tt = t[pl.ds(bb, 16)]
            w = e * tt
            t[pl.ds(bb, 16)] = tt - w
            plsc.addupdate(acc.at[pl.ds(bb, 16)], cr * w)
            plsc.addupdate(acc.at[pl.ds(bb + _PPX, 16)], cg * w)
            plsc.addupdate(acc.at[pl.ds(bb + 2 * _PPX, 16)], cb * w)
            plsc.addupdate(acc.at[pl.ds(bb + 3 * _PPX, 16)], w)

        @pl.when(jnp.logical_and(inc0, inc1))
        def _():
            @plsc.parallel_loop(ylo, yhi, unroll=2)
            def _(r):
                dy = r.astype(jnp.float32) + 0.5 - my
                ddy2 = d * dy * dy + lnop
                half_step(r, 0, u0, vv0, dy, ddy2)
                half_step(r, 16, u1, vv1, dy, ddy2)

        @pl.when(jnp.logical_and(inc0, jnp.logical_not(inc1)))
        def _():
            @plsc.parallel_loop(ylo, yhi, unroll=2)
            def _(r):
                dy = r.astype(jnp.float32) + 0.5 - my
                ddy2 = d * dy * dy + lnop
                half_step(r, 0, u0, vv0, dy, ddy2)

        @pl.when(jnp.logical_and(inc1, jnp.logical_not(inc0)))
        def _():
            @plsc.parallel_loop(ylo, yhi, unroll=2)
            def _(r):
                dy = r.astype(jnp.float32) + 0.5 - my
                ddy2 = d * dy * dy + lnop
                half_step(r, 16, u1, vv1, dy, ddy2)

        return c

    lax.fori_loop(0, cnt, hit_body, 0)

    pltpu.sync_copy(acc, out_hbm.at[wid])


def _tc_raster(n0_ref, mx_ref, my_ref, a_ref, bc_ref, d_ref,
               cr_ref, cg_ref, cb_ref, op_ref, out_ref, t_ref):
    out_ref[...] = jnp.zeros_like(out_ref)
    t_ref[...] = jnp.ones_like(t_ref)
    xs = jax.lax.broadcasted_iota(jnp.int32, (1, _W), 1).astype(jnp.float32) + 0.5
    ys = jax.lax.broadcasted_iota(jnp.int32, (_ROWS, 1), 0).astype(jnp.float32) + 0.5

    def body(i, carry):
        mx = mx_ref[i]
        my = my_ref[i]
        r0 = jnp.clip(jnp.floor(my).astype(jnp.int32) - 22, 0, _H - _ROWS)
        r0 = (r0 >> 3) << 3
        dx = xs - mx
        dy = (ys + r0.astype(jnp.float32)) - my
        q = (d_ref[i] * dy + bc_ref[i] * dx) * dy + (a_ref[i] * dx) * dx
        alpha = op_ref[i] * jnp.exp2(q)
        tt = t_ref[pl.ds(r0, _ROWS), :]
        w = alpha * tt
        t_ref[pl.ds(r0, _ROWS), :] = tt - w
        out_ref[pl.ds(r0, _ROWS), :] += cr_ref[i] * w
        out_ref[pl.ds(r0 + _H, _ROWS), :] += cg_ref[i] * w
        out_ref[pl.ds(r0 + 2 * _H, _ROWS), :] += cb_ref[i] * w
        out_ref[pl.ds(r0 + 3 * _H, _ROWS), :] += w
        return carry

    jax.lax.fori_loop(0, n0_ref[0], body, 0)


@jax.jit
def kernel(batch_ids, means, sigmas, rhos, colors, opacs):
    c = jnp.cos(rhos[:, 0])
    s = jnp.sin(rhos[:, 0])
    s1 = sigmas[:, 0]
    s2 = sigmas[:, 1]
    i1 = 1.0 / s1
    i2 = 1.0 / s2
    a = -0.5 * (c * c * i1 + s * s * i2)
    d = -0.5 * (s * s * i1 + c * c * i2)
    bc = -0.5 * (2.0 * c * s * (i1 - i2))
    sxx = c * c * s1 + s * s * s2
    syy = s * s * s1 + c * c * s2
    rx = jnp.sqrt(_QMAX * sxx)
    ry = jnp.sqrt(_QMAX * syy)
    bidf = batch_ids.astype(jnp.float32)
    lnop = jnp.log(opacs[:, 0])
    z = jnp.zeros_like(a)
    pk = jnp.stack([
        bidf, means[:, 0], means[:, 1],
        a, bc, d, colors[:, 0], colors[:, 1], colors[:, 2],
        lnop, ry, rx, z, z, z, z,
    ], axis=1).reshape(-1)
    cull = jnp.concatenate([bidf, means[:, 1], ry, means[:, 0], rx])

    out1 = _sc_render(pk, cull)
    out1 = out1.reshape(8, 4, 4, _PR, _PC)
    out1 = out1.transpose(2, 0, 3, 1, 4).reshape(4, _H, _W)

    n0 = jnp.sum(batch_ids == 0).astype(jnp.int32).reshape(1)
    smem = pl.BlockSpec(memory_space=pltpu.SMEM)
    out0 = pl.pallas_call(
        _tc_raster,
        in_specs=[smem] * 10,
        out_specs=pl.BlockSpec(memory_space=pltpu.VMEM),
        out_shape=jax.ShapeDtypeStruct((4 * _H, _W), jnp.float32),
        scratch_shapes=[pltpu.VMEM((_H, _W), jnp.float32)],
    )(n0, means[:, 0], means[:, 1], a * _LOG2E, bc * _LOG2E, d * _LOG2E,
      colors[:, 0], colors[:, 1], colors[:, 2], opacs[:, 0])
    out0 = out0.reshape(4, _H, _W)

    return jnp.stack([out0, out1], axis=0)


# speedup vs baseline: 1.3859x; 1.0345x over previous
import functools

import jax
import jax.numpy as jnp
from jax import lax
from jax.experimental import pallas as pl
from jax.experimental.pallas import tpu as pltpu
from jax.experimental.pallas import tpu_sc as plsc

_H = 128
_W = 128
_B = 2
_N = 1024
_PR = 16
_PC = 32
_PPX = _PR * _PC
_QMAX = 30.0
_ROWS = 48
_LOG2E = 1.4426950408889634

_mesh = plsc.VectorSubcoreMesh(core_axis_name="c", subcore_axis_name="s")


@functools.partial(
    pl.kernel,
    out_type=jax.ShapeDtypeStruct((32, 4 * _PPX), jnp.float32),
    mesh=_mesh,
    scratch_types=[
        pltpu.VMEM((_N * 16,), jnp.float32),
        pltpu.VMEM((5 * _N,), jnp.float32),
        pltpu.SMEM((_N + 1,), jnp.int32),
        pltpu.VMEM((16,), jnp.int32),
        pltpu.VMEM((_PPX,), jnp.float32),
        pltpu.VMEM((4 * _PPX,), jnp.float32),
    ],
)
def _sc_render(pk_hbm, cull_hbm, out_hbm, pk, cull, hits, hvbuf, t, acc):
    wid = lax.axis_index("s") * 2 + lax.axis_index("c")
    pr0 = (wid // 4) * _PR
    pc0 = (wid % 4) * _PC
    pr0f = pr0.astype(jnp.float32)
    pc0f = pc0.astype(jnp.float32)

    pltpu.sync_copy(pk_hbm, pk)
    pltpu.sync_copy(cull_hbm, cull)

    ones = jnp.full((16,), 1.0, jnp.float32)
    zeros = jnp.zeros((16,), jnp.float32)

    def init_t(j, c):
        t[pl.ds(j * 16, 16)] = ones
        return c

    lax.fori_loop(0, _PPX // 16, init_t, 0)

    def init_acc(j, c):
        acc[pl.ds(j * 16, 16)] = zeros
        return c

    lax.fori_loop(0, (4 * _PPX) // 16, init_acc, 0)

    lanei = lax.iota(jnp.int32, 16)
    lane = lanei.astype(jnp.float32) + 0.5
    x0v = lane + pc0f
    x1v = x0v + 16.0

    def cull_body(j, cnt):
        j16 = j * 16
        bidv = cull[pl.ds(j16, 16)]
        myv = cull[pl.ds(j16 + _N, 16)]
        ryv = cull[pl.ds(j16 + 2 * _N, 16)]
        mxv = cull[pl.ds(j16 + 3 * _N, 16)]
        rxv = cull[pl.ds(j16 + 4 * _N, 16)]
        hit = bidv == 1.0
        hit = jnp.logical_and(hit, myv - ryv - 0.5 <= pr0f + (_PR - 1.0))
        hit = jnp.logical_and(hit, myv + ryv + 0.5 >= pr0f)
        hit = jnp.logical_and(hit, mxv - rxv - 0.5 < pc0f + float(_PC))
        hit = jnp.logical_and(hit, mxv + rxv + 0.5 > pc0f)
        one16 = jnp.full((16,), 1, jnp.int32)
        zero16 = jnp.full((16,), 0, jnp.int32)
        hvbuf[pl.ds(0, 16)] = jnp.where(hit, one16, zero16)
        hb = hvbuf[pl.ds(0, 16)]
        base = j * 16
        for l in range(16):
            hits[cnt] = base + l
            cnt = cnt + hb[l]
        return cnt

    cnt = lax.fori_loop(0, _N // 16, cull_body, 0)

    def hit_body(k, c):
        i = hits[k]
        v = pk[pl.ds(i * 16, 16)]
        mx = v[1]
        my = v[2]
        a = v[3]
        bc = v[4]
        d = v[5]
        cr = v[6]
        cg = v[7]
        cb = v[8]
        lnop = v[9]
        ry = v[10]
        rx = v[11]
        ylo = jnp.maximum(my - ry - 0.5, pr0f).astype(jnp.int32)
        yhi = jnp.minimum(my + ry + 0.5, pr0f + (_PR - 1.0)).astype(jnp.int32) + 1
        inc0 = mx - rx - 0.5 < pc0f + 16.0
        inc1 = mx + rx + 0.5 > pc0f + 16.0

        dx0 = x0v - mx
        u0 = a * dx0 * dx0
        vv0 = bc * dx0
        dx1 = x1v - mx
        u1 = a * dx1 * dx1
        vv1 = bc * dx1

        def half_step(r, off, u, vv, dy, ddy2):
            q = (vv * dy + ddy2) + u
            e = jnp.exp(q)
            bb = (r - pr0) * _PC + off
            tt = t[pl.ds(bb, 16)]
            w = e * tt
            t[pl.ds(bb, 16)] = tt - w
            plsc.addupdate(acc.at[pl.ds(bb, 16)], cr * w)
            plsc.addupdate(acc.at[pl.ds(bb + _PPX, 16)], cg * w)
            plsc.addupdate(acc.at[pl.ds(bb + 2 * _PPX, 16)], cb * w)
            plsc.addupdate(acc.at[pl.ds(bb + 3 * _PPX, 16)], w)

        @pl.when(jnp.logical_and(inc0, inc1))
        def _():
            @plsc.parallel_loop(ylo, yhi, unroll=2)
            def _(r):
                dy = r.astype(jnp.float32) + 0.5 - my
                ddy2 = d * dy * dy + lnop
                half_step(r, 0, u0, vv0, dy, ddy2)
                half_step(r, 16, u1, vv1, dy, ddy2)

        @pl.when(jnp.logical_and(inc0, jnp.logical_not(inc1)))
        def _():
            @plsc.parallel_loop(ylo, yhi, unroll=2)
            def _(r):
                dy = r.astype(jnp.float32) + 0.5 - my
                ddy2 = d * dy * dy + lnop
                half_step(r, 0, u0, vv0, dy, ddy2)

        @pl.when(jnp.logical_and(inc1, jnp.logical_not(inc0)))
        def _():
            @plsc.parallel_loop(ylo, yhi, unroll=2)
            def _(r):
                dy = r.astype(jnp.float32) + 0.5 - my
                ddy2 = d * dy * dy + lnop
                half_step(r, 16, u1, vv1, dy, ddy2)

        return c

    lax.fori_loop(0, cnt, hit_body, 0)

    pltpu.sync_copy(acc, out_hbm.at[wid])


def _tc_raster(n0_ref, pk_ref, out_ref, t_ref):
    out_ref[...] = jnp.zeros_like(out_ref)
    t_ref[...] = jnp.ones_like(t_ref)
    xs = jax.lax.broadcasted_iota(jnp.int32, (1, _W), 1).astype(jnp.float32) + 0.5
    ys = jax.lax.broadcasted_iota(jnp.int32, (_ROWS, 1), 0).astype(jnp.float32) + 0.5

    def body(i, carry):
        ib = i * 16
        mx = pk_ref[ib + 1]
        my = pk_ref[ib + 2]
        r0 = jnp.clip(jnp.floor(my).astype(jnp.int32) - 22, 0, _H - _ROWS)
        r0 = (r0 >> 3) << 3
        dx = xs - mx
        dy = (ys + r0.astype(jnp.float32)) - my
        q = (pk_ref[ib + 5] * dy + pk_ref[ib + 4] * dx) * dy \
            + (pk_ref[ib + 3] * dx) * dx + pk_ref[ib + 9]
        w = jnp.exp(q)
        tt = t_ref[pl.ds(r0, _ROWS), :]
        w = w * tt
        t_ref[pl.ds(r0, _ROWS), :] = tt - w
        out_ref[pl.ds(r0, _ROWS), :] += pk_ref[ib + 6] * w
        out_ref[pl.ds(r0 + _H, _ROWS), :] += pk_ref[ib + 7] * w
        out_ref[pl.ds(r0 + 2 * _H, _ROWS), :] += pk_ref[ib + 8] * w
        out_ref[pl.ds(r0 + 3 * _H, _ROWS), :] += w
        return carry

    jax.lax.fori_loop(0, n0_ref[0], body, 0)


@jax.jit
def kernel(batch_ids, means, sigmas, rhos, colors, opacs):
    c = jnp.cos(rhos[:, 0])
    s = jnp.sin(rhos[:, 0])
    s1 = sigmas[:, 0]
    s2 = sigmas[:, 1]
    i1 = 1.0 / s1
    i2 = 1.0 / s2
    a = -0.5 * (c * c * i1 + s * s * i2)
    d = -0.5 * (s * s * i1 + c * c * i2)
    bc = -0.5 * (2.0 * c * s * (i1 - i2))
    sxx = c * c * s1 + s * s * s2
    syy = s * s * s1 + c * c * s2
    rx = jnp.sqrt(_QMAX * sxx)
    ry = jnp.sqrt(_QMAX * syy)
    bidf = batch_ids.astype(jnp.float32)
    lnop = jnp.log(opacs[:, 0])
    z = jnp.zeros_like(a)
    pk = jnp.stack([
        bidf, means[:, 0], means[:, 1],
        a, bc, d, colors[:, 0], colors[:, 1], colors[:, 2],
        lnop, ry, rx, z, z, z, z,
    ], axis=1).reshape(-1)
    cull = jnp.concatenate([bidf, means[:, 1], ry, means[:, 0], rx])

    out1 = _sc_render(pk, cull)
    out1 = out1.reshape(8, 4, 4, _PR, _PC)
    out1 = out1.transpose(2, 0, 3, 1, 4).reshape(4, _H, _W)

    n0 = jnp.sum(batch_ids == 0).astype(jnp.int32).reshape(1)
    smem = pl.BlockSpec(memory_space=pltpu.SMEM)
    out0 = pl.pallas_call(
        _tc_raster,
        in_specs=[smem] * 2,
        out_specs=pl.BlockSpec(memory_space=pltpu.VMEM),
        out_shape=jax.ShapeDtypeStruct((4 * _H, _W), jnp.float32),
        scratch_shapes=[pltpu.VMEM((_H, _W), jnp.float32)],
    )(n0, pk)
    out0 = out0.reshape(4, _H, _W)

    return jnp.stack([out0, out1], axis=0)
